# baseline (speedup 1.0000x reference)
import functools

import jax
import jax.numpy as jnp
from jax import lax
from jax.experimental import pallas as pl
from jax.experimental.pallas import tpu as pltpu
from jax.experimental.pallas import tpu_sc as plsc

B, T, D, S, P, R = 8, 2048, 256, 64, 2048, 16
SPAN_WIN = 32
NC, NS, L = 2, 16, 16
NW = NC * NS
CHUNK = (B * P) // NW
WPB = P // CHUNK


def _pool_project_kernel(starts_ref, lens_ref, encoded_ref, w_ref, b_ref,
                         h_ref, t_ref, pooled_ref):
    bidx = pl.program_id(0)
    neg = jnp.finfo(jnp.float32).min

    def body(s, _):
        start = starts_ref[bidx, s]
        ln = lens_ref[bidx, s]
        base = (start // 8) * 8
        off = start - base
        rows = encoded_ref[0, pl.ds(base, SPAN_WIN + 8), :]
        row_id = lax.broadcasted_iota(jnp.int32, (SPAN_WIN + 8, D), 0)
        masked = jnp.where((row_id >= off) & (row_id < off + ln), rows, neg)
        pooled_ref[pl.ds(s, 1), :] = jnp.max(masked, axis=0, keepdims=True)
        return 0

    lax.fori_loop(0, S, body, 0)

    pooled = pooled_ref[...]
    h = jnp.dot(pooled, w_ref[:D, :], preferred_element_type=jnp.float32)
    h_ref[0, :, :] = h + b_ref[...][None, :]
    t_ref[0, :, :] = jnp.dot(pooled, w_ref[D:, :],
                             preferred_element_type=jnp.float32)


def _pair_gather_kernel(h_hbm, t_hbm, head_hbm, tail_hbm, out_hbm,
                        h_v, t_v, idxh_v, idxt_v, out_v, sem):
    wid = lax.axis_index("s") * NC + lax.axis_index("c")
    base = wid * CHUNK
    bidx = wid // WPB
    tab = pl.multiple_of(bidx * S, S)
    pltpu.sync_copy(h_hbm.at[pl.ds(tab, S)], h_v)
    pltpu.sync_copy(t_hbm.at[pl.ds(tab, S)], t_v)
    pltpu.sync_copy(head_hbm.at[pl.ds(base, CHUNK)], idxh_v)
    pltpu.sync_copy(tail_hbm.at[pl.ds(base, CHUNK)], idxt_v)

    lanes = lax.iota(jnp.int32, L)
    for g in range(CHUNK // L):
        rows_h = idxh_v[pl.ds(g * L, L)]
        rows_t = idxt_v[pl.ds(g * L, L)]
        out_rows = lanes + (g * L)
        for c in range(R):
            col = jnp.full((L,), c, jnp.int32)
            vh = plsc.load_gather(h_v, [rows_h, col])
            vt = plsc.load_gather(t_v, [rows_t, col])
            plsc.store_scatter(out_v, [out_rows, col], vh + vt)

    pltpu.sync_copy(out_v, out_hbm.at[pl.ds(base, CHUNK)])


def kernel(encoded, span_starts, span_lengths, pair_head, pair_tail, W, b):
    grid_spec = pltpu.PrefetchScalarGridSpec(
        num_scalar_prefetch=2,
        grid=(B,),
        in_specs=[
            pl.BlockSpec((1, T, D), lambda b_, *_: (b_, 0, 0)),
            pl.BlockSpec((2 * D, R), lambda b_, *_: (0, 0)),
            pl.BlockSpec((R,), lambda b_, *_: (0,)),
        ],
        out_specs=[
            pl.BlockSpec((1, S, R), lambda b_, *_: (b_, 0, 0)),
            pl.BlockSpec((1, S, R), lambda b_, *_: (b_, 0, 0)),
        ],
        scratch_shapes=[pltpu.VMEM((S, D), jnp.float32)],
    )
    h, t = pl.pallas_call(
        _pool_project_kernel,
        grid_spec=grid_spec,
        out_shape=[
            jax.ShapeDtypeStruct((B, S, R), jnp.float32),
            jax.ShapeDtypeStruct((B, S, R), jnp.float32),
        ],
    )(span_starts.astype(jnp.int32), span_lengths.astype(jnp.int32),
      encoded, W, b)

    mesh = plsc.VectorSubcoreMesh(core_axis_name="c", subcore_axis_name="s")
    gather = pl.kernel(
        _pair_gather_kernel,
        out_type=jax.ShapeDtypeStruct((B * P, R), jnp.float32),
        mesh=mesh,
        compiler_params=pltpu.CompilerParams(needs_layout_passes=False),
        scratch_types=[
            pltpu.VMEM((S, R), jnp.float32),
            pltpu.VMEM((S, R), jnp.float32),
            pltpu.VMEM((CHUNK,), jnp.int32),
            pltpu.VMEM((CHUNK,), jnp.int32),
            pltpu.VMEM((CHUNK, R), jnp.float32),
            pltpu.SemaphoreType.DMA,
        ],
    )
    scores = gather(h.reshape(B * S, R), t.reshape(B * S, R),
                    pair_head.astype(jnp.int32).reshape(B * P),
                    pair_tail.astype(jnp.int32).reshape(B * P))
    return scores.reshape(B, P, R)


# speedup vs baseline: 1.0812x; 1.0812x over previous
import functools

import jax
import jax.numpy as jnp
from jax import lax
from jax.experimental import pallas as pl
from jax.experimental.pallas import tpu as pltpu
from jax.experimental.pallas import tpu_sc as plsc

B, T, D, S, P, R = 8, 2048, 256, 64, 2048, 16
SPAN_WIN = 32
NC, NS, L = 2, 16, 16
NW = NC * NS
CHUNK = (B * P) // NW
WPB = P // CHUNK


def _pool_project_kernel(starts_ref, lens_ref, encoded_ref, w_ref, b_ref,
                         h_ref, t_ref, pooled_ref):
    bidx = pl.program_id(0)
    neg = jnp.finfo(jnp.float32).min

    def body(s, _):
        start = starts_ref[bidx, s]
        ln = lens_ref[bidx, s]
        base = (start // 8) * 8
        off = start - base
        rows = encoded_ref[0, pl.ds(base, SPAN_WIN + 8), :]
        row_id = lax.broadcasted_iota(jnp.int32, (SPAN_WIN + 8, D), 0)
        masked = jnp.where((row_id >= off) & (row_id < off + ln), rows, neg)
        pooled_ref[pl.ds(s, 1), :] = jnp.max(masked, axis=0, keepdims=True)
        return 0

    lax.fori_loop(0, S, body, 0)

    pooled = pooled_ref[...]
    h = jnp.dot(pooled, w_ref[:D, :], preferred_element_type=jnp.float32)
    h_ref[0, :, :] = h + b_ref[...][None, :]
    t_ref[0, :, :] = jnp.dot(pooled, w_ref[D:, :],
                             preferred_element_type=jnp.float32)


def _pair_gather_kernel(h_hbm, t_hbm, head_hbm, tail_hbm, out_hbm,
                        h_v, t_v, idxh_v, idxt_v, out_v,
                        sem0, sem1, sem2, sem3):
    wid = lax.axis_index("s") * NC + lax.axis_index("c")
    base = wid * CHUNK
    bidx = wid // WPB
    tab = pl.multiple_of(bidx * S, S)
    cps = (pltpu.async_copy(h_hbm.at[pl.ds(tab, S)], h_v, sem0),
           pltpu.async_copy(t_hbm.at[pl.ds(tab, S)], t_v, sem1),
           pltpu.async_copy(head_hbm.at[pl.ds(base, CHUNK)], idxh_v, sem2),
           pltpu.async_copy(tail_hbm.at[pl.ds(base, CHUNK)], idxt_v, sem3))
    for cp in cps:
        cp.wait()

    lanes = lax.iota(jnp.int32, L)

    def group_body(g, _):
        off = pl.multiple_of(g * L, L)
        rows_h = idxh_v[pl.ds(off, L)]
        rows_t = idxt_v[pl.ds(off, L)]
        out_rows = lanes + g * L
        for c in range(R):
            col = jnp.full((L,), c, jnp.int32)
            vh = plsc.load_gather(h_v, [rows_h, col])
            vt = plsc.load_gather(t_v, [rows_t, col])
            plsc.store_scatter(out_v, [out_rows, col], vh + vt)
        return 0

    lax.fori_loop(0, CHUNK // L, group_body, 0)

    pltpu.sync_copy(out_v, out_hbm.at[pl.ds(base, CHUNK)])


def kernel(encoded, span_starts, span_lengths, pair_head, pair_tail, W, b):
    grid_spec = pltpu.PrefetchScalarGridSpec(
        num_scalar_prefetch=2,
        grid=(B,),
        in_specs=[
            pl.BlockSpec((1, T, D), lambda b_, *_: (b_, 0, 0)),
            pl.BlockSpec((2 * D, R), lambda b_, *_: (0, 0)),
            pl.BlockSpec((R,), lambda b_, *_: (0,)),
        ],
        out_specs=[
            pl.BlockSpec((1, S, R), lambda b_, *_: (b_, 0, 0)),
            pl.BlockSpec((1, S, R), lambda b_, *_: (b_, 0, 0)),
        ],
        scratch_shapes=[pltpu.VMEM((S, D), jnp.float32)],
    )
    h, t = pl.pallas_call(
        _pool_project_kernel,
        grid_spec=grid_spec,
        out_shape=[
            jax.ShapeDtypeStruct((B, S, R), jnp.float32),
            jax.ShapeDtypeStruct((B, S, R), jnp.float32),
        ],
    )(span_starts.astype(jnp.int32), span_lengths.astype(jnp.int32),
      encoded, W, b)

    mesh = plsc.VectorSubcoreMesh(core_axis_name="c", subcore_axis_name="s")
    gather = pl.kernel(
        _pair_gather_kernel,
        out_type=jax.ShapeDtypeStruct((B * P, R), jnp.float32),
        mesh=mesh,
        compiler_params=pltpu.CompilerParams(needs_layout_passes=False),
        scratch_types=[
            pltpu.VMEM((S, R), jnp.float32),
            pltpu.VMEM((S, R), jnp.float32),
            pltpu.VMEM((CHUNK,), jnp.int32),
            pltpu.VMEM((CHUNK,), jnp.int32),
            pltpu.VMEM((CHUNK, R), jnp.float32),
            pltpu.SemaphoreType.DMA,
            pltpu.SemaphoreType.DMA,
            pltpu.SemaphoreType.DMA,
            pltpu.SemaphoreType.DMA,
        ],
    )
    scores = gather(h.reshape(B * S, R), t.reshape(B * S, R),
                    pair_head.astype(jnp.int32).reshape(B * P),
                    pair_tail.astype(jnp.int32).reshape(B * P))
    return scores.reshape(B, P, R)


# speedup vs baseline: 1.2407x; 1.1476x over previous
import functools

import jax
import jax.numpy as jnp
from jax import lax
from jax.experimental import pallas as pl
from jax.experimental.pallas import tpu as pltpu
from jax.experimental.pallas import tpu_sc as plsc

B, T, D, S, P, R = 8, 2048, 256, 64, 2048, 16
SPAN_WIN = 32
NC, NS, L = 2, 16, 16
NW = NC * NS
CHUNK = (B * P) // NW
WPB = P // CHUNK


def _pool_project_kernel(starts_ref, lens_ref, encoded_ref, w_ref, b_ref,
                         h_ref, t_ref, pooled_ref):
    bidx = pl.program_id(0)
    neg = jnp.finfo(jnp.float32).min
    row_id = lax.broadcasted_iota(jnp.int32, (SPAN_WIN + 8, D), 0)

    for k in range(S // 8):
        group = []
        for j in range(8):
            s = k * 8 + j
            start = starts_ref[bidx, s]
            ln = lens_ref[bidx, s]
            base = (start // 8) * 8
            off = start - base
            rows = encoded_ref[0, pl.ds(base, SPAN_WIN + 8), :]
            in_span = (row_id - off).astype(jnp.uint32) < ln.astype(jnp.uint32)
            masked = jnp.where(in_span, rows, neg)
            group.append(jnp.max(masked, axis=0, keepdims=True))
        pooled_ref[k * 8:(k + 1) * 8, :] = jnp.concatenate(group, axis=0)

    pooled = pooled_ref[...]
    h = jnp.dot(pooled, w_ref[:D, :], preferred_element_type=jnp.float32)
    h_ref[0, :, :] = h + b_ref[...][None, :]
    t_ref[0, :, :] = jnp.dot(pooled, w_ref[D:, :],
                             preferred_element_type=jnp.float32)


def _pair_gather_kernel(h_hbm, t_hbm, head_hbm, tail_hbm, out_hbm,
                        h_v, t_v, idxh_v, idxt_v, out_v,
                        sem0, sem1, sem2, sem3):
    wid = lax.axis_index("s") * NC + lax.axis_index("c")
    base = wid * CHUNK
    bidx = wid // WPB
    tab = pl.multiple_of(bidx * S, S)
    cps = (pltpu.async_copy(h_hbm.at[pl.ds(tab, S)], h_v, sem0),
           pltpu.async_copy(t_hbm.at[pl.ds(tab, S)], t_v, sem1),
           pltpu.async_copy(head_hbm.at[pl.ds(base, CHUNK)], idxh_v, sem2),
           pltpu.async_copy(tail_hbm.at[pl.ds(base, CHUNK)], idxt_v, sem3))
    for cp in cps:
        cp.wait()

    lanes = lax.iota(jnp.int32, L)

    def group_body(g, _):
        off = pl.multiple_of(g * L, L)
        rows_h = idxh_v[pl.ds(off, L)]
        rows_t = idxt_v[pl.ds(off, L)]
        out_rows = lanes + g * L
        for c in range(R):
            col = jnp.full((L,), c, jnp.int32)
            vh = plsc.load_gather(h_v, [rows_h, col])
            vt = plsc.load_gather(t_v, [rows_t, col])
            plsc.store_scatter(out_v, [out_rows, col], vh + vt)
        return 0

    lax.fori_loop(0, CHUNK // L, group_body, 0)

    pltpu.sync_copy(out_v, out_hbm.at[pl.ds(base, CHUNK)])


def kernel(encoded, span_starts, span_lengths, pair_head, pair_tail, W, b):
    grid_spec = pltpu.PrefetchScalarGridSpec(
        num_scalar_prefetch=2,
        grid=(B,),
        in_specs=[
            pl.BlockSpec((1, T, D), lambda b_, *_: (b_, 0, 0)),
            pl.BlockSpec((2 * D, R), lambda b_, *_: (0, 0)),
            pl.BlockSpec((R,), lambda b_, *_: (0,)),
        ],
        out_specs=[
            pl.BlockSpec((1, S, R), lambda b_, *_: (b_, 0, 0)),
            pl.BlockSpec((1, S, R), lambda b_, *_: (b_, 0, 0)),
        ],
        scratch_shapes=[pltpu.VMEM((S, D), jnp.float32)],
    )
    h, t = pl.pallas_call(
        _pool_project_kernel,
        grid_spec=grid_spec,
        out_shape=[
            jax.ShapeDtypeStruct((B, S, R), jnp.float32),
            jax.ShapeDtypeStruct((B, S, R), jnp.float32),
        ],
    )(span_starts.astype(jnp.int32), span_lengths.astype(jnp.int32),
      encoded, W, b)

    mesh = plsc.VectorSubcoreMesh(core_axis_name="c", subcore_axis_name="s")
    gather = pl.kernel(
        _pair_gather_kernel,
        out_type=jax.ShapeDtypeStruct((B * P, R), jnp.float32),
        mesh=mesh,
        compiler_params=pltpu.CompilerParams(needs_layout_passes=False),
        scratch_types=[
            pltpu.VMEM((S, R), jnp.float32),
            pltpu.VMEM((S, R), jnp.float32),
            pltpu.VMEM((CHUNK,), jnp.int32),
            pltpu.VMEM((CHUNK,), jnp.int32),
            pltpu.VMEM((CHUNK, R), jnp.float32),
            pltpu.SemaphoreType.DMA,
            pltpu.SemaphoreType.DMA,
            pltpu.SemaphoreType.DMA,
            pltpu.SemaphoreType.DMA,
        ],
    )
    scores = gather(h.reshape(B * S, R), t.reshape(B * S, R),
                    pair_head.astype(jnp.int32).reshape(B * P),
                    pair_tail.astype(jnp.int32).reshape(B * P))
    return scores.reshape(B, P, R)


# speedup vs baseline: 1.5514x; 1.2504x over previous
import functools

import jax
import jax.numpy as jnp
from jax import lax
from jax.experimental import pallas as pl
from jax.experimental.pallas import tpu as pltpu
from jax.experimental.pallas import tpu_sc as plsc

B, T, D, S, P, R = 8, 2048, 256, 64, 2048, 16
SPAN_WIN = 32
NC, NS, L = 2, 16, 16
NW = NC * NS
CHUNK = (B * P) // NW
WPB = P // CHUNK


def _pool_project_kernel(starts_ref, lens_ref, encoded_ref, w_ref, b_ref,
                         h_ref, t_ref, pooled_ref):
    bidx = pl.program_id(0)
    neg = jnp.finfo(jnp.float32).min
    row_id = lax.broadcasted_iota(jnp.int32, (SPAN_WIN + 8, D), 0)

    for k in range(S // 8):
        group = []
        for j in range(8):
            s = k * 8 + j
            start = starts_ref[bidx, s]
            ln = lens_ref[bidx, s]
            base = (start // 8) * 8
            off = start - base
            rows = encoded_ref[0, pl.ds(base, SPAN_WIN + 8), :]
            in_span = (row_id - off).astype(jnp.uint32) < ln.astype(jnp.uint32)
            masked = jnp.where(in_span, rows, neg)
            group.append(jnp.max(masked, axis=0, keepdims=True))
        pooled_ref[k * 8:(k + 1) * 8, :] = jnp.concatenate(group, axis=0)

    pooled = pooled_ref[...]
    h = jnp.dot(pooled, w_ref[:D, :], preferred_element_type=jnp.float32)
    h_ref[0, :, :] = h + b_ref[...][None, :]
    t_ref[0, :, :] = jnp.dot(pooled, w_ref[D:, :],
                             preferred_element_type=jnp.float32)


def _pair_gather_kernel(h_hbm, t_hbm, head_hbm, tail_hbm, out_hbm,
                        h_v, t_v, idxh_v, idxt_v, out_v,
                        sem0, sem1, sem2, sem3):
    wid = lax.axis_index("s") * NC + lax.axis_index("c")
    base = wid * CHUNK
    bidx = wid // WPB
    tab = pl.multiple_of(bidx * S, S)
    cps = (pltpu.async_copy(h_hbm.at[pl.ds(tab, S)], h_v, sem0),
           pltpu.async_copy(t_hbm.at[pl.ds(tab, S)], t_v, sem1),
           pltpu.async_copy(head_hbm.at[pl.ds(base, CHUNK)], idxh_v, sem2),
           pltpu.async_copy(tail_hbm.at[pl.ds(base, CHUNK)], idxt_v, sem3))
    for cp in cps:
        cp.wait()

    def group_body(g, _):
        off = pl.multiple_of(g * L, L)
        rows_h = idxh_v[pl.ds(off, L)]
        rows_t = idxt_v[pl.ds(off, L)]
        for j in range(L):
            i = off + j
            vh = h_v[rows_h[j], :]
            vt = t_v[rows_t[j], :]
            out_v[i, :] = vh + vt
        return 0

    lax.fori_loop(0, CHUNK // L, group_body, 0)

    pltpu.sync_copy(out_v, out_hbm.at[pl.ds(base, CHUNK)])


def kernel(encoded, span_starts, span_lengths, pair_head, pair_tail, W, b):
    grid_spec = pltpu.PrefetchScalarGridSpec(
        num_scalar_prefetch=2,
        grid=(B,),
        in_specs=[
            pl.BlockSpec((1, T, D), lambda b_, *_: (b_, 0, 0)),
            pl.BlockSpec((2 * D, R), lambda b_, *_: (0, 0)),
            pl.BlockSpec((R,), lambda b_, *_: (0,)),
        ],
        out_specs=[
            pl.BlockSpec((1, S, R), lambda b_, *_: (b_, 0, 0)),
            pl.BlockSpec((1, S, R), lambda b_, *_: (b_, 0, 0)),
        ],
        scratch_shapes=[pltpu.VMEM((S, D), jnp.float32)],
    )
    h, t = pl.pallas_call(
        _pool_project_kernel,
        grid_spec=grid_spec,
        out_shape=[
            jax.ShapeDtypeStruct((B, S, R), jnp.float32),
            jax.ShapeDtypeStruct((B, S, R), jnp.float32),
        ],
    )(span_starts.astype(jnp.int32), span_lengths.astype(jnp.int32),
      encoded, W, b)

    mesh = plsc.VectorSubcoreMesh(core_axis_name="c", subcore_axis_name="s",
                                  num_cores=NC)
    gather = pl.kernel(
        _pair_gather_kernel,
        out_type=jax.ShapeDtypeStruct((B * P, R), jnp.float32),
        mesh=mesh,
        compiler_params=pltpu.CompilerParams(needs_layout_passes=False),
        scratch_types=[
            pltpu.VMEM((S, R), jnp.float32),
            pltpu.VMEM((S, R), jnp.float32),
            pltpu.VMEM((CHUNK,), jnp.int32),
            pltpu.VMEM((CHUNK,), jnp.int32),
            pltpu.VMEM((CHUNK, R), jnp.float32),
            pltpu.SemaphoreType.DMA,
            pltpu.SemaphoreType.DMA,
            pltpu.SemaphoreType.DMA,
            pltpu.SemaphoreType.DMA,
        ],
    )
    scores = gather(h.reshape(B * S, R), t.reshape(B * S, R),
                    pair_head.astype(jnp.int32).reshape(B * P),
                    pair_tail.astype(jnp.int32).reshape(B * P))
    return scores.reshape(B, P, R)


# speedup vs baseline: 1.6556x; 1.0672x over previous
import functools

import jax
import jax.numpy as jnp
from jax import lax
from jax.experimental import pallas as pl
from jax.experimental.pallas import tpu as pltpu
from jax.experimental.pallas import tpu_sc as plsc

B, T, D, S, P, R = 8, 2048, 256, 64, 2048, 16
SPAN_WIN = 32
NC, NS, L = 2, 16, 16
NW = NC * NS
CHUNK = (B * P) // NW
WPB = P // CHUNK


def _pool_project_kernel(starts_ref, lens_ref, encoded_ref, w_ref, b_ref,
                         h_ref, t_ref, pooled_ref):
    bidx = pl.program_id(0)
    neg = jnp.finfo(jnp.float32).min
    row_id = lax.broadcasted_iota(jnp.int32, (SPAN_WIN + 8, D), 0)

    for k in range(S // 8):
        group = []
        for j in range(8):
            s = k * 8 + j
            start = starts_ref[bidx, s]
            ln = lens_ref[bidx, s]
            base = (start // 8) * 8
            off = start - base
            rows = encoded_ref[0, pl.ds(base, SPAN_WIN + 8), :]
            in_span = (row_id - off).astype(jnp.uint32) < ln.astype(jnp.uint32)
            masked = jnp.where(in_span, rows, neg)
            group.append(jnp.max(masked, axis=0, keepdims=True))
        pooled_ref[k * 8:(k + 1) * 8, :] = jnp.concatenate(group, axis=0)

    pooled = pooled_ref[...]
    h = jnp.dot(pooled, w_ref[:D, :], preferred_element_type=jnp.float32)
    h_ref[0, :, :] = h + b_ref[...][None, :]
    t_ref[0, :, :] = jnp.dot(pooled, w_ref[D:, :],
                             preferred_element_type=jnp.float32)


def _pair_gather_kernel(h_hbm, t_hbm, head_hbm, tail_hbm, out_hbm,
                        h_v, t_v, idxh_v, idxt_v, out_v,
                        sem0, sem1, sem2, sem3):
    wid = lax.axis_index("s") * NC + lax.axis_index("c")
    bidx = wid // WPB
    col = pl.multiple_of((wid % WPB) * CHUNK, CHUNK)
    cps = (pltpu.async_copy(h_hbm.at[bidx], h_v, sem0),
           pltpu.async_copy(t_hbm.at[bidx], t_v, sem1),
           pltpu.async_copy(head_hbm.at[bidx, pl.ds(col, CHUNK)], idxh_v, sem2),
           pltpu.async_copy(tail_hbm.at[bidx, pl.ds(col, CHUNK)], idxt_v, sem3))
    for cp in cps:
        cp.wait()

    def group_body(g, _):
        off = pl.multiple_of(g * L, L)
        rows_h = idxh_v[pl.ds(off, L)]
        rows_t = idxt_v[pl.ds(off, L)]
        for j in range(L):
            i = off + j
            vh = h_v[rows_h[j], :]
            vt = t_v[rows_t[j], :]
            out_v[i, :] = vh + vt
        return 0

    lax.fori_loop(0, CHUNK // L, group_body, 0)

    pltpu.sync_copy(out_v, out_hbm.at[bidx, pl.ds(col, CHUNK), :])


def kernel(encoded, span_starts, span_lengths, pair_head, pair_tail, W, b):
    grid_spec = pltpu.PrefetchScalarGridSpec(
        num_scalar_prefetch=2,
        grid=(B,),
        in_specs=[
            pl.BlockSpec((1, T, D), lambda b_, *_: (b_, 0, 0)),
            pl.BlockSpec((2 * D, R), lambda b_, *_: (0, 0)),
            pl.BlockSpec((R,), lambda b_, *_: (0,)),
        ],
        out_specs=[
            pl.BlockSpec((1, S, R), lambda b_, *_: (b_, 0, 0)),
            pl.BlockSpec((1, S, R), lambda b_, *_: (b_, 0, 0)),
        ],
        scratch_shapes=[pltpu.VMEM((S, D), jnp.float32)],
    )
    h, t = pl.pallas_call(
        _pool_project_kernel,
        grid_spec=grid_spec,
        out_shape=[
            jax.ShapeDtypeStruct((B, S, R), jnp.float32),
            jax.ShapeDtypeStruct((B, S, R), jnp.float32),
        ],
    )(span_starts.astype(jnp.int32), span_lengths.astype(jnp.int32),
      encoded, W, b)

    mesh = plsc.VectorSubcoreMesh(core_axis_name="c", subcore_axis_name="s",
                                  num_cores=NC)
    gather = pl.kernel(
        _pair_gather_kernel,
        out_type=jax.ShapeDtypeStruct((B, P, R), jnp.float32),
        mesh=mesh,
        compiler_params=pltpu.CompilerParams(needs_layout_passes=False),
        scratch_types=[
            pltpu.VMEM((S, R), jnp.float32),
            pltpu.VMEM((S, R), jnp.float32),
            pltpu.VMEM((CHUNK,), jnp.int32),
            pltpu.VMEM((CHUNK,), jnp.int32),
            pltpu.VMEM((CHUNK, R), jnp.float32),
            pltpu.SemaphoreType.DMA,
            pltpu.SemaphoreType.DMA,
            pltpu.SemaphoreType.DMA,
            pltpu.SemaphoreType.DMA,
        ],
    )
    return gather(h, t, pair_head.astype(jnp.int32),
                  pair_tail.astype(jnp.int32))
